# baseline (speedup 1.0000x reference)
import functools

import jax
import jax.numpy as jnp
from jax import lax
from jax.experimental import pallas as pl
from jax.experimental.pallas import tpu as pltpu
from jax.experimental.pallas import tpu_sc as plsc

LATENT_DIM = 64
LANES = 16
CHUNK = 128


@functools.partial(jax.jit, static_argnames=())
def _mf_forward(u_idx, i_idx, users_weight, items_weight):
    batch = u_idx.shape[0] * u_idx.shape[1]
    info = plsc.get_sparse_core_info()
    nw = info.num_cores * info.num_subcores
    bpw = batch // nw
    n_chunks = bpw // CHUNK
    mesh = plsc.VectorSubcoreMesh(core_axis_name="c", subcore_axis_name="s")

    @functools.partial(
        pl.kernel,
        out_type=jax.ShapeDtypeStruct((batch,), jnp.float32),
        mesh=mesh,
        compiler_params=pltpu.CompilerParams(
            needs_layout_passes=False, use_tc_tiling_on_sc=False),
        scratch_types=[
            pltpu.VMEM((n_chunks, CHUNK), jnp.int32),
            pltpu.VMEM((n_chunks, CHUNK), jnp.int32),
            pltpu.VMEM((bpw, LATENT_DIM), jnp.float32),
            pltpu.VMEM((bpw, LATENT_DIM), jnp.float32),
            pltpu.VMEM((LANES * (LANES + 1),), jnp.float32),
            pltpu.VMEM((bpw,), jnp.float32),
            pltpu.SemaphoreType.DMA,
        ],
    )
    def kern(uidx_hbm, iidx_hbm, users_hbm, items_hbm, out_hbm,
             uidx_v, iidx_v, urows_v, irows_v, part_v, out_v, sem):
        wid = lax.axis_index("s") * info.num_cores + lax.axis_index("c")
        base = wid * bpw

        pltpu.sync_copy(uidx_hbm.at[pl.ds(wid * n_chunks, n_chunks)], uidx_v)
        pltpu.sync_copy(iidx_hbm.at[pl.ds(wid * n_chunks, n_chunks)], iidx_v)

        copies = []
        for j in range(n_chunks):
            dst = urows_v.at[pl.ds(j * CHUNK, CHUNK)]
            copies.append(pltpu.async_copy(users_hbm.at[uidx_v.at[j]], dst, sem))
            dst = irows_v.at[pl.ds(j * CHUNK, CHUNK)]
            copies.append(pltpu.async_copy(items_hbm.at[iidx_v.at[j]], dst, sem))
        for c in copies:
            c.wait()

        lanes_iota = lax.iota(jnp.int32, LANES)
        n_sub = LATENT_DIM // LANES

        def group_body(g, _):
            row0 = g * LANES
            for k in range(LANES):
                r = row0 + k
                acc = urows_v[r, pl.ds(0, LANES)] * irows_v[r, pl.ds(0, LANES)]
                for c in range(1, n_sub):
                    acc = acc + (urows_v[r, pl.ds(c * LANES, LANES)]
                                 * irows_v[r, pl.ds(c * LANES, LANES)])
                part_v[pl.ds(k * (LANES + 1), LANES)] = acc
            row_off = lanes_iota * (LANES + 1)
            tot = plsc.load_gather(part_v, [row_off])
            for l in range(1, LANES):
                tot = tot + plsc.load_gather(part_v, [row_off + l])
            out_v[pl.ds(row0, LANES)] = tot
            return 0

        lax.fori_loop(0, bpw // LANES, group_body, 0)

        pltpu.sync_copy(out_v, out_hbm.at[pl.ds(base, bpw)])

    return kern(u_idx, i_idx, users_weight, items_weight)


def kernel(x, users_weight, items_weight):
    x32 = x.astype(jnp.int32)
    u_idx = x32[:, 0].reshape(-1, CHUNK)
    i_idx = x32[:, 1].reshape(-1, CHUNK)
    return _mf_forward(u_idx, i_idx, users_weight, items_weight)
